# baseline (speedup 1.0000x reference)
import functools

import jax
import jax.numpy as jnp
from jax import lax
from jax.experimental import pallas as pl
from jax.experimental.pallas import tpu as pltpu
from jax.experimental.pallas import tpu_sc as plsc


def _gather_call(N, DP, b_per_w, C, nbuf, mesh, num_cores):
    n_chunks = b_per_w // C
    assert n_chunks % nbuf == 0 and n_chunks >= 2 * nbuf

    @functools.partial(
        pl.kernel,
        mesh=mesh,
        out_type=jax.ShapeDtypeStruct((N, DP), jnp.float32),
        scratch_types=[
            pltpu.VMEM((b_per_w,), jnp.int32),
            pltpu.VMEM((nbuf, C, DP), jnp.float32),
            pltpu.SemaphoreType.DMA((nbuf,)),
            pltpu.SemaphoreType.DMA((nbuf,)),
        ],
    )
    def k(idx_hbm, tbl_hbm, out_hbm, idx_v, rows_v, gsem, osem):
        wid = lax.axis_index("s") * num_cores + lax.axis_index("c")
        base = wid * b_per_w
        pltpu.sync_copy(idx_hbm.at[pl.ds(base, b_per_w)], idx_v)

        def gdesc(c, b):
            return pltpu.make_async_copy(
                tbl_hbm.at[idx_v.at[pl.ds(c * C, C)]], rows_v.at[b], gsem.at[b]
            )

        def odesc(c, b):
            return pltpu.make_async_copy(
                rows_v.at[b], out_hbm.at[pl.ds(base + c * C, C)], osem.at[b]
            )

        for b in range(nbuf):
            gdesc(b, b).start()

        def body(i, carry):
            i0 = i * nbuf
            for b in range(nbuf):
                gdesc(i0 + b, b).wait()
                odesc(i0 + b, b).start()
            for b in range(nbuf):
                odesc(i0 + b, b).wait()
                gdesc(i0 + b + nbuf, b).start()
            return carry

        lax.fori_loop(0, (n_chunks - nbuf) // nbuf, body, 0)

        c0 = n_chunks - nbuf
        for b in range(nbuf):
            gdesc(c0 + b, b).wait()
            odesc(c0 + b, b).start()
        for b in range(nbuf):
            odesc(c0 + b, b).wait()

    return k


def _pad_call(V, D, DP, Rv):
    def body(t_ref, o_ref):
        o_ref[:, :D] = t_ref[...]
        o_ref[:, D:] = jnp.zeros((Rv, DP - D), jnp.float32)

    return pl.pallas_call(
        body,
        grid=(V // Rv,),
        in_specs=[pl.BlockSpec((Rv, D), lambda i: (i, 0))],
        out_specs=pl.BlockSpec((Rv, DP), lambda i: (i, 0)),
        out_shape=jax.ShapeDtypeStruct((V, DP), jnp.float32),
    )


def _compact_call(B, H, D, DP, Rb):
    def body(g_ref, o_ref):
        o_ref[...] = g_ref[...].reshape(Rb, H, DP)[:, :, :D]

    return pl.pallas_call(
        body,
        grid=(B // Rb,),
        in_specs=[pl.BlockSpec((Rb * H, DP), lambda i: (i, 0))],
        out_specs=pl.BlockSpec((Rb, H, D), lambda i: (i, 0, 0)),
        out_shape=jax.ShapeDtypeStruct((B, H, D), jnp.float32),
    )


def kernel(x, table):
    B, H = x.shape
    V, D = table.shape
    DP = 128
    N = B * H
    idx = x.reshape(N).astype(jnp.int32)
    tbl = _pad_call(V, D, DP, 8000)(table)

    info = plsc.get_sparse_core_info()
    num_workers = info.num_cores * info.num_subcores
    b_per_w = N // num_workers

    mesh = plsc.VectorSubcoreMesh(core_axis_name="c", subcore_axis_name="s")
    g = _gather_call(N, DP, b_per_w, 256, 2, mesh, info.num_cores)(idx, tbl)
    return _compact_call(B, H, D, DP, 64)(g)


# speedup vs baseline: 1.2918x; 1.2918x over previous
import functools

import jax
import jax.numpy as jnp
from jax import lax
from jax.experimental import pallas as pl
from jax.experimental.pallas import tpu as pltpu
from jax.experimental.pallas import tpu_sc as plsc


def _gather_call(B, H, D, b_per_w, C, nbuf, mesh, num_cores):
    n_chunks = b_per_w // C
    rows_per_chunk = C // H
    assert C % H == 0 and n_chunks % nbuf == 0 and n_chunks >= 2 * nbuf

    @functools.partial(
        pl.kernel,
        mesh=mesh,
        compiler_params=pltpu.CompilerParams(use_tc_tiling_on_sc=False),
        out_type=jax.ShapeDtypeStruct((B, H, D), jnp.float32),
        scratch_types=[
            pltpu.VMEM((b_per_w,), jnp.int32),
            pltpu.VMEM((nbuf, C, D), jnp.float32),
            pltpu.SemaphoreType.DMA((nbuf,)),
            pltpu.SemaphoreType.DMA((nbuf,)),
        ],
    )
    def k(idx_hbm, tbl_hbm, out_hbm, idx_v, rows_v, gsem, osem):
        wid = lax.axis_index("s") * num_cores + lax.axis_index("c")
        base = wid * b_per_w
        row_base = base // H
        pltpu.sync_copy(idx_hbm.at[pl.ds(base, b_per_w)], idx_v)

        def gdesc(c, b):
            return pltpu.make_async_copy(
                tbl_hbm.at[idx_v.at[pl.ds(c * C, C)]], rows_v.at[b], gsem.at[b]
            )

        def odesc(c, b, j):
            return pltpu.make_async_copy(
                rows_v.at[b, pl.ds(j * H, H), pl.ds(0, D)],
                out_hbm.at[row_base + c * rows_per_chunk + j],
                osem.at[b],
            )

        for b in range(nbuf):
            gdesc(b, b).start()

        def body(i, carry):
            i0 = i * nbuf
            for b in range(nbuf):
                gdesc(i0 + b, b).wait()
                for j in range(rows_per_chunk):
                    odesc(i0 + b, b, j).start()
            for b in range(nbuf):
                for j in range(rows_per_chunk):
                    odesc(i0 + b, b, j).wait()
                gdesc(i0 + b + nbuf, b).start()
            return carry

        lax.fori_loop(0, (n_chunks - nbuf) // nbuf, body, 0)

        c0 = n_chunks - nbuf
        for b in range(nbuf):
            gdesc(c0 + b, b).wait()
            for j in range(rows_per_chunk):
                odesc(c0 + b, b, j).start()
        for b in range(nbuf):
            for j in range(rows_per_chunk):
                odesc(c0 + b, b, j).wait()

    return k


def kernel(x, table):
    B, H = x.shape
    V, D = table.shape
    N = B * H
    idx = x.reshape(N).astype(jnp.int32)

    info = plsc.get_sparse_core_info()
    num_workers = info.num_cores * info.num_subcores
    b_per_w = N // num_workers

    mesh = plsc.VectorSubcoreMesh(core_axis_name="c", subcore_axis_name="s")
    return _gather_call(B, H, D, b_per_w, 400, 2, mesh, info.num_cores)(
        idx, table
    )
